# baseline (speedup 1.0000x reference)
import functools

import jax
import jax.numpy as jnp
import numpy as np
from jax import lax
from jax.experimental import pallas as pl
from jax.experimental.pallas import tpu as pltpu
from jax.experimental.pallas import tpu_sc as plsc

F = 26
D = 32
B = 1024
VB = 1000
VH = 504
L = 16

NC, NS = 2, 16
NW = NC * NS
NG = B // L

_II, _JJ = np.triu_indices(F, k=1)
N_UNITS = 2 * len(_II) + F
NU_PAD = ((N_UNITS + NW - 1) // NW) * NW
NU_BASE = N_UNITS // NW
NU_EXTRA = N_UNITS - NU_BASE * NW


def _build_meta() -> np.ndarray:
    m = np.zeros((NU_PAD, 16), np.int32)
    u = 0
    for i, j in zip(_II, _JJ):
        for h in range(2):
            m[u, :9] = (j, i * VB + h * 496, h * 496, 0,
                        i, j * VB, i, j, h * 500)
            u += 1
    for f in range(F):
        m[u, :9] = (0, 0, 0, 1, 0, 0, f, 0, 0)
        u += 1
    m[u:, 3] = 1
    return m


_META_NP = _build_meta()


def _ffm_body(meta_hbm, xt_hbm, wl_hbm, table_hbm, part_hbm,
              meta_v, abuf0, abuf1, bbuf0, bbuf1, wband0, wband1,
              xca0, xcb0, xca1, xcb1, acc_v, sem0, sem1):
    wid = lax.axis_index("s") * NC + lax.axis_index("c")
    sems = (sem0, sem1)
    abufs = (abuf0, abuf1)
    bbufs = (bbuf0, bbuf1)
    wbands = (wband0, wband1)
    xcas = (xca0, xca1)
    xcbs = (xcb0, xcb1)

    pltpu.sync_copy(meta_hbm, meta_v)

    zero = jnp.zeros((L,), jnp.float32)

    def zbody(g, c):
        acc_v[0, g] = zero
        return c

    lax.fori_loop(0, NG, zbody, 0, unroll=False)

    def refs(u, slot):
        mrow = meta_v[u]
        ta = mrow[0]
        va = pl.multiple_of(mrow[1], 8)
        tb = mrow[4]
        vb = pl.multiple_of(mrow[5], 8)
        ca = pl.multiple_of(mrow[6] * B, B)
        cb = pl.multiple_of(mrow[7] * B, B)
        fw = pl.multiple_of(mrow[6] * 1024, 1024)
        sem = sems[slot]
        pair_copies = (
            (table_hbm.at[pl.ds(ta, 1), pl.ds(va, VH), :], abufs[slot]),
            (table_hbm.at[pl.ds(tb, 1), pl.ds(vb, VB), :], bbufs[slot]),
            (xt_hbm.at[pl.ds(ca, B)], xcas[slot]),
            (xt_hbm.at[pl.ds(cb, B)], xcbs[slot]),
        )
        lin_copies = (
            (wl_hbm.at[pl.ds(fw, 1024)], wbands[slot]),
            (xt_hbm.at[pl.ds(ca, B)], xcas[slot]),
        )
        return mrow[3], pair_copies, lin_copies, sem

    def issue(u, slot):
        kind, pair_copies, lin_copies, sem = refs(u, slot)

        @pl.when(kind == 0)
        def _():
            for s, d in pair_copies:
                pltpu.async_copy(s, d, sem)

        @pl.when(kind == 1)
        def _():
            for s, d in lin_copies:
                pltpu.async_copy(s, d, sem)

    def drain(u, slot):
        kind, pair_copies, lin_copies, sem = refs(u, slot)

        @pl.when(kind == 0)
        def _():
            for s, d in pair_copies:
                pltpu.make_async_copy(s, d, sem).wait()

        @pl.when(kind == 1)
        def _():
            for s, d in lin_copies:
                pltpu.make_async_copy(s, d, sem).wait()

    dconsts = [jnp.full((L,), d, jnp.int32) for d in range(D)]

    def compute(u, slot):
        mrow = meta_v[u]
        kind = mrow[3]
        rel_off = mrow[2]
        lo = mrow[8]

        @pl.when(kind == 0)
        def _():
            ra = abufs[slot]
            rb = bbufs[slot]
            zv = jnp.zeros((L,), jnp.int32)

            def gbody(g, c):
                xi = xcas[slot][pl.ds(g * L, L)]
                xj = xcbs[slot][pl.ds(g * L, L)]
                active = jnp.logical_and(xi >= lo, xi < lo + 500)
                relc = jnp.where(active, xi - rel_off, 0)
                dot = jnp.zeros((L,), jnp.float32)
                for d in range(D):
                    av = plsc.load_gather(ra, [zv, relc, dconsts[d]])
                    bv = plsc.load_gather(rb, [zv, xj, dconsts[d]])
                    dot = dot + av * bv
                acc_v[0, g] = acc_v[0, g] + jnp.where(active, dot, 0.0)
                return c

            lax.fori_loop(0, NG, gbody, 0, unroll=False)

        @pl.when(kind == 1)
        def _():
            wb = wbands[slot]

            def gbody(g, c):
                xf = xcas[slot][pl.ds(g * L, L)]
                acc_v[0, g] = acc_v[0, g] + plsc.load_gather(wb, [xf])
                return c

            lax.fori_loop(0, NG, gbody, 0, unroll=False)

    nu = NU_BASE + (wid < NU_EXTRA).astype(jnp.int32)

    issue(wid, 0)

    def body(s, c):
        u = wid + NW * s
        unext = u + NW

        @pl.when(jnp.logical_and(s + 1 < nu, (s + 1) % 2 == 0))
        def _():
            issue(unext, 0)

        @pl.when(jnp.logical_and(s + 1 < nu, (s + 1) % 2 == 1))
        def _():
            issue(unext, 1)

        @pl.when(s % 2 == 0)
        def _():
            drain(u, 0)
            compute(u, 0)

        @pl.when(s % 2 == 1)
        def _():
            drain(u, 1)
            compute(u, 1)

        return c

    lax.fori_loop(0, nu, body, 0, unroll=False)

    pltpu.sync_copy(acc_v, part_hbm.at[pl.ds(wid, 1)])


def _combine_body(part_hbm, bias_hbm, out_hbm, pall_v, bias_v, ob_v):
    wid = lax.axis_index("s") * NC + lax.axis_index("c")
    pltpu.sync_copy(part_hbm, pall_v)
    pltpu.sync_copy(bias_hbm, bias_v)
    bias_vec = bias_v[...]
    for r in range(2):
        row = 2 * wid + r

        def sbody(k, acc):
            return acc + pall_v[k, row]

        s = lax.fori_loop(0, NW, sbody, bias_vec, unroll=False)
        ob_v[0, r] = 1.0 / (1.0 + jnp.exp(-s))
    pltpu.sync_copy(ob_v, out_hbm.at[pl.ds(wid, 1)])


_SC_PARAMS = pltpu.CompilerParams(
    use_tc_tiling_on_sc=False, needs_layout_passes=False)
_MESH = dict(mesh=plsc.VectorSubcoreMesh(core_axis_name="c",
                                         subcore_axis_name="s"))


@jax.jit
def _ffm_sc(meta, xt, wl_pad, bias_bcast, table3):
    part = functools.partial(
        pl.kernel,
        out_type=jax.ShapeDtypeStruct((NW, NG, L), jnp.float32),
        compiler_params=_SC_PARAMS,
        scratch_types=[
            pltpu.VMEM((NU_PAD, 16), jnp.int32),
            pltpu.VMEM((1, VH, D), jnp.float32),
            pltpu.VMEM((1, VH, D), jnp.float32),
            pltpu.VMEM((1, VB, D), jnp.float32),
            pltpu.VMEM((1, VB, D), jnp.float32),
            pltpu.VMEM((1024,), jnp.float32),
            pltpu.VMEM((1024,), jnp.float32),
            pltpu.VMEM((B,), jnp.int32),
            pltpu.VMEM((B,), jnp.int32),
            pltpu.VMEM((B,), jnp.int32),
            pltpu.VMEM((B,), jnp.int32),
            pltpu.VMEM((1, NG, L), jnp.float32),
            pltpu.SemaphoreType.DMA,
            pltpu.SemaphoreType.DMA,
        ],
        **_MESH,
    )(_ffm_body)(meta, xt, wl_pad, table3)

    out = functools.partial(
        pl.kernel,
        out_type=jax.ShapeDtypeStruct((NW, 2, L), jnp.float32),
        compiler_params=_SC_PARAMS,
        scratch_types=[
            pltpu.VMEM((NW, NG, L), jnp.float32),
            pltpu.VMEM((L,), jnp.float32),
            pltpu.VMEM((1, 2, L), jnp.float32),
        ],
        **_MESH,
    )(_combine_body)(part, bias_bcast)
    return out


def kernel(x, W_lin, bias, ffm_tables):
    meta = jnp.asarray(_META_NP)
    xt = x.T.reshape(-1)
    wl_pad = jnp.pad(W_lin.reshape(F, VB), ((0, 0), (0, 1024 - VB))).reshape(-1)
    out = _ffm_sc(meta, xt, wl_pad,
                  jnp.broadcast_to(bias, (L,)), ffm_tables)
    return out.reshape(-1)


# speedup vs baseline: 1.3928x; 1.3928x over previous
import functools

import jax
import jax.numpy as jnp
import numpy as np
from jax import lax
from jax.experimental import pallas as pl
from jax.experimental.pallas import tpu as pltpu
from jax.experimental.pallas import tpu_sc as plsc

F = 26
D = 32
DH = 16
B = 1024
VB = 1000
L = 16

NC, NS = 2, 16
NW = NC * NS
NG = B // L

_II, _JJ = np.triu_indices(F, k=1)
N_UNITS = 2 * len(_II) + F
NU_PAD = ((N_UNITS + NW - 1) // NW) * NW
NU_BASE = N_UNITS // NW
NU_EXTRA = N_UNITS - NU_BASE * NW


def _build_meta() -> np.ndarray:
    m = np.zeros((NU_PAD, 16), np.int32)
    u = 0
    for i, j in zip(_II, _JJ):
        for dh in range(2):
            m[u, :8] = (j, i * VB, dh, 0, i, j * VB, i, j)
            u += 1
    for f in range(F):
        m[u, :8] = (0, f * VB, 0, 1, 0, 0, f, 0)
        u += 1
    m[u:, 3] = 1
    return m


_META_NP = _build_meta()


def _ffm_body(meta_hbm, xt_hbm, wl_hbm, table_hbm, part_hbm,
              meta_v, rows_v, wband_v, xcol_v, acc_v, sem0, sem1):
    wid = lax.axis_index("s") * NC + lax.axis_index("c")
    sems = (sem0, sem1)

    pltpu.sync_copy(meta_hbm, meta_v)

    zero = jnp.zeros((L,), jnp.float32)

    def zbody(g, c):
        acc_v[g] = zero
        return c

    lax.fori_loop(0, NG, zbody, 0, unroll=False)

    def issue(u, slot):
        mrow = meta_v[u]
        kind = mrow[3]
        sem = sems[slot]

        @pl.when(kind == 0)
        def _():
            ta = mrow[0]
            va = pl.multiple_of(mrow[1], 8)
            d0 = mrow[2]
            tb = mrow[4]
            vb = pl.multiple_of(mrow[5], 8)
            ca, cb = mrow[6], mrow[7]
            pltpu.async_copy(table_hbm.at[d0, ta, pl.ds(va, VB), :],
                             rows_v.at[2 * slot], sem)
            pltpu.async_copy(table_hbm.at[d0, tb, pl.ds(vb, VB), :],
                             rows_v.at[2 * slot + 1], sem)
            pltpu.async_copy(xt_hbm.at[ca], xcol_v.at[slot, 0], sem)
            pltpu.async_copy(xt_hbm.at[cb], xcol_v.at[slot, 1], sem)

        @pl.when(kind == 1)
        def _():
            va, ca = pl.multiple_of(mrow[1], 8), mrow[6]
            pltpu.async_copy(wl_hbm.at[pl.ds(va, VB)], wband_v.at[slot], sem)
            pltpu.async_copy(xt_hbm.at[ca], xcol_v.at[slot, 0], sem)

    def drain(u, slot):
        mrow = meta_v[u]
        kind = mrow[3]
        sem = sems[slot]

        @pl.when(kind == 0)
        def _():
            ta = mrow[0]
            va = pl.multiple_of(mrow[1], 8)
            d0 = mrow[2]
            tb = mrow[4]
            vb = pl.multiple_of(mrow[5], 8)
            ca, cb = mrow[6], mrow[7]
            pltpu.make_async_copy(table_hbm.at[d0, ta, pl.ds(va, VB), :],
                                  rows_v.at[2 * slot], sem).wait()
            pltpu.make_async_copy(table_hbm.at[d0, tb, pl.ds(vb, VB), :],
                                  rows_v.at[2 * slot + 1], sem).wait()
            pltpu.make_async_copy(xt_hbm.at[ca], xcol_v.at[slot, 0], sem).wait()
            pltpu.make_async_copy(xt_hbm.at[cb], xcol_v.at[slot, 1], sem).wait()

        @pl.when(kind == 1)
        def _():
            va, ca = pl.multiple_of(mrow[1], 8), mrow[6]
            pltpu.make_async_copy(wl_hbm.at[pl.ds(va, VB)], wband_v.at[slot],
                                  sem).wait()
            pltpu.make_async_copy(xt_hbm.at[ca], xcol_v.at[slot, 0], sem).wait()

    dconsts = [jnp.full((L,), d, jnp.int32) for d in range(DH)]

    def compute(u, slot):
        kind = meta_v[u][3]

        @pl.when(kind == 0)
        def _():
            ra = rows_v.at[2 * slot]
            rb = rows_v.at[2 * slot + 1]

            def gbody(g, c):
                xi = xcol_v[slot, 0, pl.ds(g * L, L)]
                xj = xcol_v[slot, 1, pl.ds(g * L, L)]
                accg = acc_v[g]
                for d in range(DH):
                    av = plsc.load_gather(ra, [xi, dconsts[d]])
                    bv = plsc.load_gather(rb, [xj, dconsts[d]])
                    accg = accg + av * bv
                acc_v[g] = accg
                return c

            lax.fori_loop(0, NG, gbody, 0, unroll=False)

        @pl.when(kind == 1)
        def _():
            wb = wband_v.at[slot]

            def gbody(g, c):
                xf = xcol_v[slot, 0, pl.ds(g * L, L)]
                acc_v[g] = acc_v[g] + plsc.load_gather(wb, [xf])
                return c

            lax.fori_loop(0, NG, gbody, 0, unroll=False)

    nu = NU_BASE + (wid < NU_EXTRA).astype(jnp.int32)

    issue(wid, 0)

    def body(s, c):
        u = wid + NW * s
        unext = u + NW

        @pl.when(jnp.logical_and(s + 1 < nu, (s + 1) % 2 == 0))
        def _():
            issue(unext, 0)

        @pl.when(jnp.logical_and(s + 1 < nu, (s + 1) % 2 == 1))
        def _():
            issue(unext, 1)

        @pl.when(s % 2 == 0)
        def _():
            drain(u, 0)
            compute(u, 0)

        @pl.when(s % 2 == 1)
        def _():
            drain(u, 1)
            compute(u, 1)

        return c

    lax.fori_loop(0, nu, body, 0, unroll=False)

    pltpu.sync_copy(acc_v, part_hbm.at[wid])


def _combine_body(part_hbm, bias_hbm, out_hbm, pall_v, bias_v, ob_v):
    wid = lax.axis_index("s") * NC + lax.axis_index("c")
    pltpu.sync_copy(part_hbm, pall_v)
    pltpu.sync_copy(bias_hbm, bias_v)
    bias_vec = bias_v[...]
    for r in range(2):
        row = 2 * wid + r

        def sbody(k, acc):
            return acc + pall_v[k, row]

        s = lax.fori_loop(0, NW, sbody, bias_vec, unroll=False)
        ob_v[r] = 1.0 / (1.0 + jnp.exp(-s))
    pltpu.sync_copy(ob_v, out_hbm.at[pl.ds(2 * wid, 2)])


_SC_PARAMS = pltpu.CompilerParams(
    use_tc_tiling_on_sc=False, needs_layout_passes=False)
_MESH = dict(mesh=plsc.VectorSubcoreMesh(core_axis_name="c",
                                         subcore_axis_name="s"))


@jax.jit
def _ffm_sc(meta, xt, wl_flat, bias_bcast, table3):
    part = functools.partial(
        pl.kernel,
        out_type=jax.ShapeDtypeStruct((NW, NG, L), jnp.float32),
        compiler_params=_SC_PARAMS,
        scratch_types=[
            pltpu.VMEM((NU_PAD, 16), jnp.int32),
            pltpu.VMEM((4, VB, DH), jnp.float32),
            pltpu.VMEM((2, VB), jnp.float32),
            pltpu.VMEM((2, 2, B), jnp.int32),
            pltpu.VMEM((NG, L), jnp.float32),
            pltpu.SemaphoreType.DMA,
            pltpu.SemaphoreType.DMA,
        ],
        **_MESH,
    )(_ffm_body)(meta, xt, wl_flat, table3)

    out = functools.partial(
        pl.kernel,
        out_type=jax.ShapeDtypeStruct((NG, L), jnp.float32),
        compiler_params=_SC_PARAMS,
        scratch_types=[
            pltpu.VMEM((NW, NG, L), jnp.float32),
            pltpu.VMEM((L,), jnp.float32),
            pltpu.VMEM((2, L), jnp.float32),
        ],
        **_MESH,
    )(_combine_body)(part, bias_bcast)
    return out


def kernel(x, W_lin, bias, ffm_tables):
    meta = jnp.asarray(_META_NP)
    xt = x.T
    tds = ffm_tables.reshape(F, 26000, 2, DH).transpose(2, 0, 1, 3)
    out = _ffm_sc(meta, xt, W_lin.reshape(-1),
                  jnp.broadcast_to(bias, (L,)), tds)
    return out.reshape(-1)


# speedup vs baseline: 4.7699x; 3.4246x over previous
import functools

import jax
import jax.numpy as jnp
import numpy as np
from jax import lax
from jax.experimental import pallas as pl
from jax.experimental.pallas import tpu as pltpu
from jax.experimental.pallas import tpu_sc as plsc

F = 26
D = 32
DH = 16
B = 1024
VB = 1000
L = 16

NC, NS = 2, 16
NW = NC * NS
NG = B // L

_II, _JJ = np.triu_indices(F, k=1)
N_UNITS = 2 * len(_II) + F
NU_PAD = ((N_UNITS + NW - 1) // NW) * NW
NU_BASE = N_UNITS // NW
NU_EXTRA = N_UNITS - NU_BASE * NW


def _build_meta() -> np.ndarray:
    m = np.zeros((NU_PAD, 16), np.int32)
    u = 0
    for i, j in zip(_II, _JJ):
        for dh in range(2):
            m[u, :8] = (j, i * VB, dh * DH, 0, i, j * VB, i, j)
            u += 1
    for f in range(F):
        m[u, :8] = (0, f * VB, 0, 1, 0, 0, f, 0)
        u += 1
    m[u:, 3] = 1
    return m


_META_NP = _build_meta()


def _ffm_body(meta_hbm, xt_hbm, wl_hbm, table_hbm, part_hbm,
              meta_v, rows_v, wband_v, xcol_v, acc_v, sem0, sem1):
    wid = lax.axis_index("s") * NC + lax.axis_index("c")
    sems = (sem0, sem1)

    pltpu.sync_copy(meta_hbm, meta_v)

    zero = jnp.zeros((L,), jnp.float32)

    def zbody(g, c):
        acc_v[g] = zero
        return c

    lax.fori_loop(0, NG, zbody, 0, unroll=False)

    def issue(u, slot):
        mrow = meta_v[u]
        kind = mrow[3]
        sem = sems[slot]

        @pl.when(kind == 0)
        def _():
            ta = mrow[0]
            va = pl.multiple_of(mrow[1], 8)
            d0 = pl.multiple_of(mrow[2], 16)
            tb = mrow[4]
            vb = pl.multiple_of(mrow[5], 8)
            ca, cb = mrow[6], mrow[7]
            pltpu.async_copy(table_hbm.at[ta, pl.ds(d0, DH), pl.ds(va, VB)],
                             rows_v.at[2 * slot], sem)
            pltpu.async_copy(table_hbm.at[tb, pl.ds(d0, DH), pl.ds(vb, VB)],
                             rows_v.at[2 * slot + 1], sem)
            pltpu.async_copy(xt_hbm.at[ca], xcol_v.at[slot, 0], sem)
            pltpu.async_copy(xt_hbm.at[cb], xcol_v.at[slot, 1], sem)

        @pl.when(kind == 1)
        def _():
            va, ca = pl.multiple_of(mrow[1], 8), mrow[6]
            pltpu.async_copy(wl_hbm.at[pl.ds(va, VB)], wband_v.at[slot], sem)
            pltpu.async_copy(xt_hbm.at[ca], xcol_v.at[slot, 0], sem)

    def drain(u, slot):
        mrow = meta_v[u]
        kind = mrow[3]
        sem = sems[slot]

        @pl.when(kind == 0)
        def _():
            ta = mrow[0]
            va = pl.multiple_of(mrow[1], 8)
            d0 = pl.multiple_of(mrow[2], 16)
            tb = mrow[4]
            vb = pl.multiple_of(mrow[5], 8)
            ca, cb = mrow[6], mrow[7]
            pltpu.make_async_copy(table_hbm.at[ta, pl.ds(d0, DH), pl.ds(va, VB)],
                                  rows_v.at[2 * slot], sem).wait()
            pltpu.make_async_copy(table_hbm.at[tb, pl.ds(d0, DH), pl.ds(vb, VB)],
                                  rows_v.at[2 * slot + 1], sem).wait()
            pltpu.make_async_copy(xt_hbm.at[ca], xcol_v.at[slot, 0], sem).wait()
            pltpu.make_async_copy(xt_hbm.at[cb], xcol_v.at[slot, 1], sem).wait()

        @pl.when(kind == 1)
        def _():
            va, ca = pl.multiple_of(mrow[1], 8), mrow[6]
            pltpu.make_async_copy(wl_hbm.at[pl.ds(va, VB)], wband_v.at[slot],
                                  sem).wait()
            pltpu.make_async_copy(xt_hbm.at[ca], xcol_v.at[slot, 0], sem).wait()

    dconsts = [jnp.full((L,), d, jnp.int32) for d in range(DH)]

    def compute(u, slot):
        kind = meta_v[u][3]

        @pl.when(kind == 0)
        def _():
            ra = rows_v.at[2 * slot]
            rb = rows_v.at[2 * slot + 1]

            def gbody(g, c):
                xi = xcol_v[slot, 0, pl.ds(g * L, L)]
                xj = xcol_v[slot, 1, pl.ds(g * L, L)]
                accg = acc_v[g]
                for d in range(DH):
                    av = plsc.load_gather(ra, [dconsts[d], xi])
                    bv = plsc.load_gather(rb, [dconsts[d], xj])
                    accg = accg + av * bv
                acc_v[g] = accg
                return c

            lax.fori_loop(0, NG, gbody, 0, unroll=False)

        @pl.when(kind == 1)
        def _():
            wb = wband_v.at[slot]

            def gbody(g, c):
                xf = xcol_v[slot, 0, pl.ds(g * L, L)]
                acc_v[g] = acc_v[g] + plsc.load_gather(wb, [xf])
                return c

            lax.fori_loop(0, NG, gbody, 0, unroll=False)

    nu = NU_BASE + (wid < NU_EXTRA).astype(jnp.int32)

    issue(wid, 0)

    def body(s, c):
        u = wid + NW * s
        unext = u + NW

        @pl.when(jnp.logical_and(s + 1 < nu, (s + 1) % 2 == 0))
        def _():
            issue(unext, 0)

        @pl.when(jnp.logical_and(s + 1 < nu, (s + 1) % 2 == 1))
        def _():
            issue(unext, 1)

        @pl.when(s % 2 == 0)
        def _():
            drain(u, 0)
            compute(u, 0)

        @pl.when(s % 2 == 1)
        def _():
            drain(u, 1)
            compute(u, 1)

        return c

    lax.fori_loop(0, nu, body, 0, unroll=False)

    pltpu.sync_copy(acc_v, part_hbm.at[wid])


def _combine_body(part_hbm, bias_hbm, out_hbm, pall_v, bias_v, ob_v):
    wid = lax.axis_index("s") * NC + lax.axis_index("c")
    pltpu.sync_copy(part_hbm, pall_v)
    pltpu.sync_copy(bias_hbm, bias_v)
    bias_vec = bias_v[...]
    for r in range(2):
        row = 2 * wid + r

        def sbody(k, acc):
            return acc + pall_v[k, row]

        s = lax.fori_loop(0, NW, sbody, bias_vec, unroll=False)
        ob_v[r] = 1.0 / (1.0 + jnp.exp(-s))
    pltpu.sync_copy(ob_v, out_hbm.at[pl.ds(2 * wid, 2)])


_SC_PARAMS = pltpu.CompilerParams(
    use_tc_tiling_on_sc=False, needs_layout_passes=False)
_MESH = dict(mesh=plsc.VectorSubcoreMesh(core_axis_name="c",
                                         subcore_axis_name="s"))


@jax.jit
def _ffm_sc(meta, xt, wl_flat, bias_bcast, table3):
    part = functools.partial(
        pl.kernel,
        out_type=jax.ShapeDtypeStruct((NW, NG, L), jnp.float32),
        compiler_params=_SC_PARAMS,
        scratch_types=[
            pltpu.VMEM((NU_PAD, 16), jnp.int32),
            pltpu.VMEM((4, DH, VB), jnp.float32),
            pltpu.VMEM((2, VB), jnp.float32),
            pltpu.VMEM((2, 2, B), jnp.int32),
            pltpu.VMEM((NG, L), jnp.float32),
            pltpu.SemaphoreType.DMA,
            pltpu.SemaphoreType.DMA,
        ],
        **_MESH,
    )(_ffm_body)(meta, xt, wl_flat, table3)

    out = functools.partial(
        pl.kernel,
        out_type=jax.ShapeDtypeStruct((NG, L), jnp.float32),
        compiler_params=_SC_PARAMS,
        scratch_types=[
            pltpu.VMEM((NW, NG, L), jnp.float32),
            pltpu.VMEM((L,), jnp.float32),
            pltpu.VMEM((2, L), jnp.float32),
        ],
        **_MESH,
    )(_combine_body)(part, bias_bcast)
    return out


def kernel(x, W_lin, bias, ffm_tables):
    meta = jnp.asarray(_META_NP)
    xt = x.T
    tt = ffm_tables.transpose(0, 2, 1)
    out = _ffm_sc(meta, xt, W_lin.reshape(-1),
                  jnp.broadcast_to(bias, (L,)), tt)
    return out.reshape(-1)


# speedup vs baseline: 10.5752x; 2.2171x over previous
import functools

import jax
import jax.numpy as jnp
import numpy as np
from jax import lax
from jax.experimental import pallas as pl
from jax.experimental.pallas import tpu as pltpu
from jax.experimental.pallas import tpu_sc as plsc

F = 26
D = 32
DH = 16
B = 1024
VB = 1000
VW = 1152
L = 16

NC, NS = 2, 16
NW = NC * NS
NG = B // L

_II, _JJ = np.triu_indices(F, k=1)
N_UNITS = 2 * len(_II) + F
NU_PAD = ((N_UNITS + NW - 1) // NW) * NW
NU_BASE = N_UNITS // NW
NU_EXTRA = N_UNITS - NU_BASE * NW


def _build_meta() -> np.ndarray:
    m = np.zeros((NU_PAD, 16), np.int32)
    u = 0
    for i, j in zip(_II, _JJ):
        for dh in range(2):
            va, vb = i * VB, j * VB
            vaal, vbal = (va // 128) * 128, (vb // 128) * 128
            m[u, :9] = (j * D + dh * DH, vaal, va - vaal, 0,
                        i * D + dh * DH, vbal, i, j, vb - vbal)
            u += 1
    for f in range(F):
        m[u, :9] = (0, 0, 0, 1, 0, 0, f, 0, 0)
        u += 1
    m[u:, 3] = 1
    return m


_META_NP = _build_meta()


def _ffm_body(meta_hbm, xt_hbm, wl_hbm, table_hbm, part_hbm,
              meta_v, ab0, ab1, bb0, bb1, wband0, wband1,
              xca0, xcb0, xca1, xcb1, acc_v, sem0, sem1):
    wid = lax.axis_index("s") * NC + lax.axis_index("c")
    sems = (sem0, sem1)
    abufs = (ab0, ab1)
    bbufs = (bb0, bb1)
    wbands = (wband0, wband1)
    xcas = (xca0, xca1)
    xcbs = (xcb0, xcb1)

    pltpu.sync_copy(meta_hbm, meta_v)

    zero = jnp.zeros((L,), jnp.float32)

    def zbody(g, c):
        acc_v[pl.ds(pl.multiple_of(g * L, L), L)] = zero
        return c

    lax.fori_loop(0, NG, zbody, 0, unroll=False)

    def refs(u, slot):
        mrow = meta_v[pl.ds(pl.multiple_of(u * 16, 8), 16)]
        ra = pl.multiple_of(mrow[0], DH)
        va = pl.multiple_of(mrow[1], 128)
        rb = pl.multiple_of(mrow[4], DH)
        vb = pl.multiple_of(mrow[5], 128)
        ca = pl.multiple_of(mrow[6] * B, B)
        cb = pl.multiple_of(mrow[7] * B, B)
        fw = pl.multiple_of(mrow[6] * 1024, 1024)
        sem = sems[slot]
        pair_copies = (
            (table_hbm.at[pl.ds(ra, DH), pl.ds(va, VW)], abufs[slot]),
            (table_hbm.at[pl.ds(rb, DH), pl.ds(vb, VW)], bbufs[slot]),
            (xt_hbm.at[pl.ds(ca, B)], xcas[slot]),
            (xt_hbm.at[pl.ds(cb, B)], xcbs[slot]),
        )
        lin_copies = (
            (wl_hbm.at[pl.ds(fw, 1024)], wbands[slot]),
            (xt_hbm.at[pl.ds(ca, B)], xcas[slot]),
        )
        return mrow[3], pair_copies, lin_copies, sem

    def issue(u, slot):
        kind, pair_copies, lin_copies, sem = refs(u, slot)

        @pl.when(kind == 0)
        def _():
            for s, d in pair_copies:
                pltpu.async_copy(s, d, sem)

        @pl.when(kind == 1)
        def _():
            for s, d in lin_copies:
                pltpu.async_copy(s, d, sem)

    def drain(u, slot):
        kind, pair_copies, lin_copies, sem = refs(u, slot)

        @pl.when(kind == 0)
        def _():
            for s, d in pair_copies:
                pltpu.make_async_copy(s, d, sem).wait()

        @pl.when(kind == 1)
        def _():
            for s, d in lin_copies:
                pltpu.make_async_copy(s, d, sem).wait()

    dconsts = [jnp.full((L,), d, jnp.int32) for d in range(DH)]

    def compute(u, slot):
        mrow = meta_v[pl.ds(pl.multiple_of(u * 16, 8), 16)]
        kind = mrow[3]
        offa = mrow[2]
        offb = mrow[8]

        @pl.when(kind == 0)
        def _():
            ra = abufs[slot]
            rb = bbufs[slot]

            def gbody(g, c):
                xi = xcas[slot][pl.ds(pl.multiple_of(g * L, L), L)] + offa
                xj = xcbs[slot][pl.ds(pl.multiple_of(g * L, L), L)] + offb
                goff = pl.multiple_of(g * L, L)
                accg = acc_v[pl.ds(goff, L)]
                for d in range(DH):
                    av = plsc.load_gather(ra, [dconsts[d], xi])
                    bv = plsc.load_gather(rb, [dconsts[d], xj])
                    accg = accg + av * bv
                acc_v[pl.ds(goff, L)] = accg
                return c

            lax.fori_loop(0, NG, gbody, 0, unroll=False)

        @pl.when(kind == 1)
        def _():
            wb = wbands[slot]

            def gbody(g, c):
                goff = pl.multiple_of(g * L, L)
                xf = xcas[slot][pl.ds(goff, L)]
                acc_v[pl.ds(goff, L)] = (acc_v[pl.ds(goff, L)]
                                         + plsc.load_gather(wb, [xf]))
                return c

            lax.fori_loop(0, NG, gbody, 0, unroll=False)

    nu = NU_BASE + (wid < NU_EXTRA).astype(jnp.int32)

    issue(wid, 0)

    def body(s, c):
        u = wid + NW * s
        unext = u + NW

        @pl.when(jnp.logical_and(s + 1 < nu, (s + 1) % 2 == 0))
        def _():
            issue(unext, 0)

        @pl.when(jnp.logical_and(s + 1 < nu, (s + 1) % 2 == 1))
        def _():
            issue(unext, 1)

        @pl.when(s % 2 == 0)
        def _():
            drain(u, 0)
            compute(u, 0)

        @pl.when(s % 2 == 1)
        def _():
            drain(u, 1)
            compute(u, 1)

        return c

    lax.fori_loop(0, nu, body, 0, unroll=False)

    pltpu.sync_copy(acc_v, part_hbm.at[pl.ds(pl.multiple_of(wid * B, B), B)])


def _combine_body(part_hbm, bias_hbm, out_hbm, pall_v, bias_v, ob_v):
    wid = lax.axis_index("s") * NC + lax.axis_index("c")
    pltpu.sync_copy(part_hbm, pall_v)
    pltpu.sync_copy(bias_hbm, bias_v)
    bias_vec = bias_v[...]
    for r in range(2):
        row = 2 * wid + r

        def sbody(k, acc):
            return acc + pall_v[pl.ds(pl.multiple_of(k * B + row * L, 8), L)]

        s = lax.fori_loop(0, NW, sbody, bias_vec, unroll=False)
        ob_v[0, r] = 1.0 / (1.0 + jnp.exp(-s))
    pltpu.sync_copy(ob_v, out_hbm.at[pl.ds(wid, 1)])


_SC_PARAMS = pltpu.CompilerParams(
    use_tc_tiling_on_sc=True, needs_layout_passes=False)
_MESH = dict(mesh=plsc.VectorSubcoreMesh(core_axis_name="c",
                                         subcore_axis_name="s"))


@jax.jit
def _ffm_sc(meta, xt, wl_pad, bias_bcast, table2):
    part = functools.partial(
        pl.kernel,
        out_type=jax.ShapeDtypeStruct((NW * B,), jnp.float32),
        compiler_params=_SC_PARAMS,
        scratch_types=[
            pltpu.VMEM((NU_PAD * 16,), jnp.int32),
            pltpu.VMEM((DH, VW), jnp.float32),
            pltpu.VMEM((DH, VW), jnp.float32),
            pltpu.VMEM((DH, VW), jnp.float32),
            pltpu.VMEM((DH, VW), jnp.float32),
            pltpu.VMEM((1024,), jnp.float32),
            pltpu.VMEM((1024,), jnp.float32),
            pltpu.VMEM((B,), jnp.int32),
            pltpu.VMEM((B,), jnp.int32),
            pltpu.VMEM((B,), jnp.int32),
            pltpu.VMEM((B,), jnp.int32),
            pltpu.VMEM((B,), jnp.float32),
            pltpu.SemaphoreType.DMA,
            pltpu.SemaphoreType.DMA,
        ],
        **_MESH,
    )(_ffm_body)(meta, xt, wl_pad, table2)

    out = functools.partial(
        pl.kernel,
        out_type=jax.ShapeDtypeStruct((NW, 2, L), jnp.float32),
        compiler_params=_SC_PARAMS,
        scratch_types=[
            pltpu.VMEM((NW * B,), jnp.float32),
            pltpu.VMEM((L,), jnp.float32),
            pltpu.VMEM((1, 2, L), jnp.float32),
        ],
        **_MESH,
    )(_combine_body)(part, bias_bcast)
    return out


def kernel(x, W_lin, bias, ffm_tables):
    meta = jnp.asarray(_META_NP.reshape(-1))
    xt = x.T.reshape(-1)
    wl_pad = jnp.pad(W_lin.reshape(F, VB), ((0, 0), (0, 1024 - VB))).reshape(-1)
    tt = ffm_tables.transpose(0, 2, 1).reshape(F * D, W_lin.shape[0])
    out = _ffm_sc(meta, xt, wl_pad,
                  jnp.broadcast_to(bias, (L,)), tt)
    return out.reshape(-1)
